# baseline (device time: 259512 ns/iter reference)
import jax
import jax.numpy as jnp
from jax import lax
from jax.experimental import pallas as pl
from jax.experimental.pallas import tpu as pltpu

N = 16
ROWS = 1024
COLS = 512
D_ROWS = 8
D_COLS = 128


def _body(x_ref, d_ref, out_x, out_d, x_send, x_recv, d_send, d_recv):
    me = lax.axis_index("i")
    left = lax.rem(me + (N - 1), N)
    right = lax.rem(me + 1, N)

    barrier = pltpu.get_barrier_semaphore()
    pl.semaphore_signal(barrier, inc=1, device_id=(left,),
                        device_id_type=pl.DeviceIdType.MESH)
    pl.semaphore_signal(barrier, inc=1, device_id=(right,),
                        device_id_type=pl.DeviceIdType.MESH)
    pl.semaphore_wait(barrier, 2)

    out_x[pl.ds(me * ROWS, ROWS), :] = x_ref[...]
    out_d[pl.ds(me * D_ROWS, D_ROWS), :] = d_ref[...]

    sends = []
    recvs = []
    for h in range(N - 1):
        cs = lax.rem(me - h + N, N)
        cr = lax.rem(me - 1 - h + 2 * N, N)
        send_x = pltpu.make_async_remote_copy(
            src_ref=out_x.at[pl.ds(cs * ROWS, ROWS), :],
            dst_ref=out_x.at[pl.ds(cs * ROWS, ROWS), :],
            send_sem=x_send.at[h], recv_sem=x_recv.at[h],
            device_id=(right,), device_id_type=pl.DeviceIdType.MESH,
        )
        send_d = pltpu.make_async_remote_copy(
            src_ref=out_d.at[pl.ds(cs * D_ROWS, D_ROWS), :],
            dst_ref=out_d.at[pl.ds(cs * D_ROWS, D_ROWS), :],
            send_sem=d_send.at[h], recv_sem=d_recv.at[h],
            device_id=(right,), device_id_type=pl.DeviceIdType.MESH,
        )
        recv_x = pltpu.make_async_remote_copy(
            src_ref=out_x.at[pl.ds(cr * ROWS, ROWS), :],
            dst_ref=out_x.at[pl.ds(cr * ROWS, ROWS), :],
            send_sem=x_send.at[h], recv_sem=x_recv.at[h],
            device_id=(left,), device_id_type=pl.DeviceIdType.MESH,
        )
        recv_d = pltpu.make_async_remote_copy(
            src_ref=out_d.at[pl.ds(cr * D_ROWS, D_ROWS), :],
            dst_ref=out_d.at[pl.ds(cr * D_ROWS, D_ROWS), :],
            send_sem=d_send.at[h], recv_sem=d_recv.at[h],
            device_id=(left,), device_id_type=pl.DeviceIdType.MESH,
        )
        if h > 0:
            recvs[h - 1][0].wait_recv()
            recvs[h - 1][1].wait_recv()
        send_x.start()
        send_d.start()
        sends.append((send_x, send_d))
        recvs.append((recv_x, recv_d))

    recvs[-1][0].wait_recv()
    recvs[-1][1].wait_recv()
    for sx, sd in sends:
        sx.wait_send()
        sd.wait_send()


def kernel(x, dest):
    x_bf = x.astype(jnp.bfloat16)
    d2 = dest.reshape(D_ROWS, D_COLS)

    gx, gd = pl.pallas_call(
        _body,
        out_shape=[
            jax.ShapeDtypeStruct((N * ROWS, COLS), jnp.bfloat16),
            jax.ShapeDtypeStruct((N * D_ROWS, D_COLS), jnp.int32),
        ],
        in_specs=[
            pl.BlockSpec(memory_space=pltpu.VMEM),
            pl.BlockSpec(memory_space=pltpu.VMEM),
        ],
        out_specs=[
            pl.BlockSpec(memory_space=pltpu.VMEM),
            pl.BlockSpec(memory_space=pltpu.VMEM),
        ],
        scratch_shapes=[
            pltpu.SemaphoreType.DMA((N - 1,)),
            pltpu.SemaphoreType.DMA((N - 1,)),
            pltpu.SemaphoreType.DMA((N - 1,)),
            pltpu.SemaphoreType.DMA((N - 1,)),
        ],
        compiler_params=pltpu.CompilerParams(collective_id=0),
    )(x_bf, d2)

    me = lax.axis_index("i")
    idx = jnp.nonzero(gd.reshape(-1) == me, size=ROWS)[0]
    return gx[idx].astype(jnp.float32)
